# baseline (speedup 1.0000x reference)
import jax
import jax.numpy as jnp
from jax.experimental import pallas as pl
from jax.experimental.pallas import tpu as pltpu

_B, _N, _M = 8, 2048, 2048
_K = 16
_S = _B + 1


def _bf16_split3(x, guard=False):
    barrier = jax.lax.optimization_barrier if guard else (lambda v: v)
    hi = x.astype(jnp.bfloat16)
    r1 = barrier(x - hi.astype(jnp.float32))
    mid = r1.astype(jnp.bfloat16)
    r2 = barrier(r1 - mid.astype(jnp.float32))
    lo = r2.astype(jnp.bfloat16)
    return hi, mid, lo


def _augment_a(p1):
    a16 = p1.astype(jnp.bfloat16)
    a2 = jnp.sum(p1 * p1, axis=2, keepdims=True)
    a2h, a2m, a2l = _bf16_split3(a2, guard=True)
    ones_a = jnp.ones_like(a2, dtype=jnp.bfloat16)
    zeros_a = jnp.zeros(a2.shape[:2] + (_K - 9,), dtype=jnp.bfloat16)
    return jnp.concatenate(
        [a16, a2h, a2m, a2l, ones_a, ones_a, ones_a, zeros_a], axis=2)


def _chamfer_tc_kernel(a_ref, bt_ref, out_ref, dbuf_ref):
    s = pl.program_id(0)

    d = dbuf_ref[(s - 1) % 2]
    rowmin = jnp.min(d, axis=1)
    colmin = jnp.min(d, axis=0)
    contrib = (jnp.sum(rowmin) * (1.0 / (_B * _N))
               + jnp.sum(colmin) * (1.0 / (_B * _M)))
    out_ref[0] = jnp.where(s > 0, out_ref[0] + jnp.where(s > 0, contrib, 0.0),
                           0.0)

    bt = bt_ref[0]
    bx, by, bz = bt[0:1, :], bt[1:2, :], bt[2:3, :]
    b2 = bx * bx + by * by + bz * bz
    b2h, b2m, b2l = _bf16_split3(b2)
    m2 = jnp.float32(-2.0)
    onesb = jnp.ones_like(b2, dtype=jnp.bfloat16)
    zerosb = jnp.zeros((_K - 9, _M), dtype=jnp.bfloat16)
    b_aug = jnp.concatenate(
        [(m2 * bx.astype(jnp.bfloat16).astype(jnp.float32)).astype(jnp.bfloat16),
         (m2 * by.astype(jnp.bfloat16).astype(jnp.float32)).astype(jnp.bfloat16),
         (m2 * bz.astype(jnp.bfloat16).astype(jnp.float32)).astype(jnp.bfloat16),
         onesb, onesb, onesb, b2h, b2m, b2l, zerosb], axis=0)

    dbuf_ref[s % 2] = jax.lax.dot_general(
        a_ref[0], b_aug, (((1,), (0,)), ((), ())),
        preferred_element_type=jnp.float32)


def kernel(p1, p2):
    a_aug = _augment_a(p1)
    p2t = jnp.transpose(p2, (0, 2, 1))
    out = pl.pallas_call(
        _chamfer_tc_kernel,
        grid=(_S,),
        in_specs=[
            pl.BlockSpec((1, _N, _K), lambda s: (jnp.minimum(s, _B - 1), 0, 0)),
            pl.BlockSpec((1, 3, _M), lambda s: (jnp.minimum(s, _B - 1), 0, 0)),
        ],
        out_specs=pl.BlockSpec(memory_space=pltpu.SMEM),
        out_shape=jax.ShapeDtypeStruct((1,), jnp.float32),
        scratch_shapes=[pltpu.VMEM((2, _N, _M), jnp.float32)],
    )(a_aug, p2t)
    return out


# speedup vs baseline: 1.2906x; 1.2906x over previous
import jax
import jax.numpy as jnp
from jax.experimental import pallas as pl
from jax.experimental.pallas import tpu as pltpu

_B, _N, _M = 8, 2048, 2048
_K = 16
_S = _B + 1


def _bf16_split3(x, guard=False):
    barrier = jax.lax.optimization_barrier if guard else (lambda v: v)
    hi = x.astype(jnp.bfloat16)
    r1 = barrier(x - hi.astype(jnp.float32))
    mid = r1.astype(jnp.bfloat16)
    r2 = barrier(r1 - mid.astype(jnp.float32))
    lo = r2.astype(jnp.bfloat16)
    return hi, mid, lo


def _augment_a(p1):
    a16 = p1.astype(jnp.bfloat16)
    a2 = jnp.sum(p1 * p1, axis=2, keepdims=True)
    a2h, a2m, a2l = _bf16_split3(a2, guard=True)
    ones_a = jnp.ones_like(a2, dtype=jnp.bfloat16)
    zeros_a = jnp.zeros(a2.shape[:2] + (_K - 9,), dtype=jnp.bfloat16)
    return jnp.concatenate(
        [a16, a2h, a2m, a2l, ones_a, ones_a, ones_a, zeros_a], axis=2)


def _chamfer_tc_kernel(a_ref, bt_ref, out_ref, dbufa_ref, dbufb_ref):
    s = pl.program_id(0)

    def stage(dcur_ref, dprev_ref):
        d = dprev_ref[...]
        rowmin = jnp.min(d, axis=1)
        colmin = jnp.min(d, axis=0)
        contrib = (jnp.sum(rowmin) * (1.0 / (_B * _N))
                   + jnp.sum(colmin) * (1.0 / (_B * _M)))
        out_ref[0] = jnp.where(
            s > 0, out_ref[0] + jnp.where(s > 0, contrib, 0.0), 0.0)

        bt = bt_ref[0]
        bx, by, bz = bt[0:1, :], bt[1:2, :], bt[2:3, :]
        b2 = bx * bx + by * by + bz * bz
        b2h, b2m, b2l = _bf16_split3(b2)
        m2 = jnp.float32(-2.0)
        onesb = jnp.ones_like(b2, dtype=jnp.bfloat16)
        zerosb = jnp.zeros((_K - 9, _M), dtype=jnp.bfloat16)
        b_aug = jnp.concatenate(
            [(m2 * bx.astype(jnp.bfloat16).astype(jnp.float32)).astype(jnp.bfloat16),
             (m2 * by.astype(jnp.bfloat16).astype(jnp.float32)).astype(jnp.bfloat16),
             (m2 * bz.astype(jnp.bfloat16).astype(jnp.float32)).astype(jnp.bfloat16),
             onesb, onesb, onesb, b2h, b2m, b2l, zerosb], axis=0)

        dcur_ref[...] = jax.lax.dot_general(
            a_ref[0], b_aug, (((1,), (0,)), ((), ())),
            preferred_element_type=jnp.float32)

    @pl.when(s % 2 == 0)
    def _():
        stage(dbufa_ref, dbufb_ref)

    @pl.when(s % 2 == 1)
    def _():
        stage(dbufb_ref, dbufa_ref)


def kernel(p1, p2):
    a_aug = _augment_a(p1)
    p2t = jnp.transpose(p2, (0, 2, 1))
    out = pl.pallas_call(
        _chamfer_tc_kernel,
        grid=(_S,),
        in_specs=[
            pl.BlockSpec((1, _N, _K), lambda s: (jnp.minimum(s, _B - 1), 0, 0)),
            pl.BlockSpec((1, 3, _M), lambda s: (jnp.minimum(s, _B - 1), 0, 0)),
        ],
        out_specs=pl.BlockSpec(memory_space=pltpu.SMEM),
        out_shape=jax.ShapeDtypeStruct((1,), jnp.float32),
        scratch_shapes=[pltpu.VMEM((_N, _M), jnp.float32),
                        pltpu.VMEM((_N, _M), jnp.float32)],
    )(a_aug, p2t)
    return out


# speedup vs baseline: 1.3972x; 1.0826x over previous
import jax
import jax.numpy as jnp
from jax.experimental import pallas as pl
from jax.experimental.pallas import tpu as pltpu

_B, _N, _M = 8, 2048, 2048
_K = 16
_S = _B + 1


def _bf16_split3(x, guard=False):
    barrier = jax.lax.optimization_barrier if guard else (lambda v: v)
    hi = x.astype(jnp.bfloat16)
    r1 = barrier(x - hi.astype(jnp.float32))
    mid = r1.astype(jnp.bfloat16)
    r2 = barrier(r1 - mid.astype(jnp.float32))
    lo = r2.astype(jnp.bfloat16)
    return hi, mid, lo


def _augment_a(p1):
    a16 = p1.astype(jnp.bfloat16)
    a2 = jnp.sum(p1 * p1, axis=2, keepdims=True)
    a2h, a2m, a2l = _bf16_split3(a2, guard=True)
    ones_a = jnp.ones_like(a2, dtype=jnp.bfloat16)
    zeros_a = jnp.zeros(a2.shape[:2] + (_K - 9,), dtype=jnp.bfloat16)
    return jnp.concatenate(
        [a16, a2h, a2m, a2l, ones_a, ones_a, ones_a, zeros_a], axis=2)


def _chamfer_tc_kernel(a_ref, bt_ref, out_ref):
    s = pl.program_id(0)

    bt = bt_ref[0]
    bx, by, bz = bt[0:1, :], bt[1:2, :], bt[2:3, :]
    b2 = bx * bx + by * by + bz * bz
    b2h, b2m, b2l = _bf16_split3(b2)
    m2 = jnp.float32(-2.0)
    onesb = jnp.ones_like(b2, dtype=jnp.bfloat16)
    zerosb = jnp.zeros((_K - 9, _M), dtype=jnp.bfloat16)
    b_aug = jnp.concatenate(
        [(m2 * bx.astype(jnp.bfloat16).astype(jnp.float32)).astype(jnp.bfloat16),
         (m2 * by.astype(jnp.bfloat16).astype(jnp.float32)).astype(jnp.bfloat16),
         (m2 * bz.astype(jnp.bfloat16).astype(jnp.float32)).astype(jnp.bfloat16),
         onesb, onesb, onesb, b2h, b2m, b2l, zerosb], axis=0)

    d = jax.lax.dot_general(
        a_ref[0], b_aug, (((1,), (0,)), ((), ())),
        preferred_element_type=jnp.float32)

    rowmin = jnp.min(d, axis=1)
    colmin = jnp.min(d, axis=0)
    contrib = (jnp.sum(rowmin) * (1.0 / (_B * _N))
               + jnp.sum(colmin) * (1.0 / (_B * _M)))

    @pl.when(s == 0)
    def _():
        out_ref[0] = 0.0

    out_ref[0] += contrib


def kernel(p1, p2):
    a_aug = _augment_a(p1)
    p2t = jnp.transpose(p2, (0, 2, 1))
    out = pl.pallas_call(
        _chamfer_tc_kernel,
        grid=(_B,),
        in_specs=[
            pl.BlockSpec((1, _N, _K), lambda s: (s, 0, 0)),
            pl.BlockSpec((1, 3, _M), lambda s: (s, 0, 0)),
        ],
        out_specs=pl.BlockSpec(memory_space=pltpu.SMEM),
        out_shape=jax.ShapeDtypeStruct((1,), jnp.float32),
    )(a_aug, p2t)
    return out
